# baseline (speedup 1.0000x reference)
import functools

import jax
import jax.numpy as jnp
from jax import lax
from jax.experimental import pallas as pl
from jax.experimental.pallas import tpu as pltpu
from jax.experimental.pallas import tpu_sc as plsc

N = 50000
E = 800000
H = 4
C = 32
HC = 128

NC = 2
NS = 16
NW = NC * NS
W_PER = E // NW
CH = 1000
CHUNKS = W_PER // CH
FULLG = CH // 16
IDXW = 125
NP = 50048
ZR = NP // NS

_mesh = plsc.VectorSubcoreMesh(
    core_axis_name="c", subcore_axis_name="s", num_cores=NC, num_subcores=NS)
_sc_params = pltpu.CompilerParams(
    needs_layout_passes=False, use_tc_tiling_on_sc=False)


@functools.partial(
    pl.kernel,
    out_type=jax.ShapeDtypeStruct((4, E), jnp.float32),
    mesh=_mesh,
    scratch_types=[
        pltpu.VMEM((2 * N,), jnp.float32),
        pltpu.VMEM((2, CH), jnp.int32),
        pltpu.VMEM((2, CH), jnp.int32),
        pltpu.VMEM((2, 4, CH), jnp.float32),
        pltpu.SemaphoreType.DMA,
        pltpu.SemaphoreType.DMA,
        pltpu.SemaphoreType.DMA,
        pltpu.SemaphoreType.DMA,
        pltpu.SemaphoreType.DMA,
    ],
    compiler_params=_sc_params,
)
def _sc_gather(x_hbm, src_hbm, dst_hbm, out_hbm, xtab, sbuf, dbuf, cbuf,
               sx, si0, si1, so0, so1):
    c = lax.axis_index("c")
    s = lax.axis_index("s")
    wid = s * NC + c
    base = wid * W_PER
    s_in = (si0, si1)
    s_out = (so0, so1)

    xc = pltpu.async_copy(x_hbm, xtab, sx)

    def start_in(ci):
        b = ci & 1
        off = pl.multiple_of(base + ci * CH, CH)
        d1 = pltpu.async_copy(src_hbm.at[pl.ds(off, CH)], sbuf.at[b], s_in[b])
        d2 = pltpu.async_copy(dst_hbm.at[pl.ds(off, CH)], dbuf.at[b], s_in[b])
        return (d1, d2)

    in_d = [None, None]
    in_d[0] = start_in(0)
    xc.wait()
    out_d = [None, None]

    for ci in range(CHUNKS):
        b = ci & 1
        if ci + 1 < CHUNKS:
            in_d[1 - b] = start_in(ci + 1)
        in_d[b][0].wait()
        in_d[b][1].wait()
        if out_d[b] is not None:
            for d in out_d[b]:
                d.wait()

        @pl.loop(0, FULLG)
        def _grp(g, b=b):
            si = sbuf[b, pl.ds(g * 16, 16)] * 2
            di = dbuf[b, pl.ds(g * 16, 16)] * 2
            cbuf[b, 0, pl.ds(g * 16, 16)] = plsc.load_gather(xtab, [si])
            cbuf[b, 1, pl.ds(g * 16, 16)] = plsc.load_gather(xtab, [si + 1])
            cbuf[b, 2, pl.ds(g * 16, 16)] = plsc.load_gather(xtab, [di])
            cbuf[b, 3, pl.ds(g * 16, 16)] = plsc.load_gather(xtab, [di + 1])

        to = CH - 16
        sit = sbuf[b, pl.ds(to, 16)] * 2
        dit = dbuf[b, pl.ds(to, 16)] * 2
        cbuf[b, 0, pl.ds(to, 16)] = plsc.load_gather(xtab, [sit])
        cbuf[b, 1, pl.ds(to, 16)] = plsc.load_gather(xtab, [sit + 1])
        cbuf[b, 2, pl.ds(to, 16)] = plsc.load_gather(xtab, [dit])
        cbuf[b, 3, pl.ds(to, 16)] = plsc.load_gather(xtab, [dit + 1])

        off = pl.multiple_of(base + ci * CH, CH)
        out_d[b] = [
            pltpu.async_copy(cbuf.at[b, r], out_hbm.at[r, pl.ds(off, CH)],
                             s_out[b])
            for r in range(4)]

    for bb in range(2):
        if out_d[bb] is not None:
            for d in out_d[bb]:
                d.wait()


@functools.partial(
    pl.kernel,
    out_type=jax.ShapeDtypeStruct((NC, NP, 16), jnp.float32),
    mesh=_mesh,
    scratch_types=[
        pltpu.VMEM_SHARED((NP, 16), jnp.float32),
        pltpu.VMEM((CHUNKS * 8, IDXW), jnp.int32),
        pltpu.VMEM((2, 4, CH), jnp.float32),
        pltpu.VMEM((2, 2, CH), jnp.float32),
        pltpu.VMEM((2, CH, 16), jnp.float32),
        pltpu.SemaphoreType.DMA,
        pltpu.SemaphoreType.DMA,
        pltpu.SemaphoreType.DMA,
        pltpu.SemaphoreType.DMA,
    ],
    compiler_params=_sc_params,
)
def _sc_scatter(ex_hbm, co_hbm, dst_hbm, out_hbm,
                tsh, dbuf, exbuf, abbuf, vbuf, si0, si1, sc0, sc1):
    c = lax.axis_index("c")
    s = lax.axis_index("s")
    wid = s * NC + c
    base = wid * W_PER
    s_in = (si0, si1)
    s_sc = (sc0, sc1)

    dst_pf = pltpu.async_copy(
        dst_hbm.at[pl.ds(pl.multiple_of(base // IDXW, 8), CHUNKS * 8)],
        dbuf, si0)

    @pl.loop(0, CH)
    def _z(i):
        vbuf[0, i, :] = jnp.zeros((16,), jnp.float32)
        vbuf[1, i, :] = jnp.zeros((16,), jnp.float32)

    row0 = pl.multiple_of(s * ZR, 8)
    for r in range(3):
        pltpu.sync_copy(vbuf.at[0],
                        tsh.at[pl.ds(row0 + r * CH, CH)])
    pltpu.sync_copy(vbuf.at[0, pl.ds(0, ZR - 3 * CH)],
                    tsh.at[pl.ds(row0 + 3 * CH, ZR - 3 * CH)])
    plsc.subcore_barrier()

    def start_in(ci):
        b = ci & 1
        off = pl.multiple_of(base + ci * CH, CH)
        ds = [pltpu.async_copy(ex_hbm.at[h, pl.ds(off, CH)],
                               exbuf.at[b, h], s_in[b]) for h in range(4)]
        ds += [pltpu.async_copy(co_hbm.at[r, pl.ds(off, CH)],
                                abbuf.at[b, r], s_in[b]) for r in range(2)]
        return ds

    in_d = [None, None]
    in_d[0] = start_in(0)
    dst_pf.wait()
    sc_d = [[], []]

    for ci in range(CHUNKS):
        b = ci & 1
        if ci + 1 < CHUNKS:
            in_d[1 - b] = start_in(ci + 1)
        for d in in_d[b]:
            d.wait()
        for d in sc_d[b]:
            d.wait()
        sc_d[b] = []

        @pl.loop(0, FULLG)
        def _grp(g, b=b):
            rows = g * 16 + lax.iota(jnp.int32, 16)
            a = abbuf[b, 0, pl.ds(g * 16, 16)]
            bb = abbuf[b, 1, pl.ds(g * 16, 16)]
            for h in range(H):
                exh = exbuf[b, h, pl.ds(g * 16, 16)]
                col = jnp.full((16,), h, jnp.int32)
                plsc.store_scatter(vbuf.at[b], [rows, col], exh)
                plsc.store_scatter(vbuf.at[b], [rows, col + 4], exh * a)
                plsc.store_scatter(vbuf.at[b], [rows, col + 8], exh * bb)

        to = CH - 16
        rows_t = to + lax.iota(jnp.int32, 16)
        mask_t = lax.iota(jnp.int32, 16) >= 8
        a_t = abbuf[b, 0, pl.ds(to, 16)]
        b_t = abbuf[b, 1, pl.ds(to, 16)]
        for h in range(H):
            exh = exbuf[b, h, pl.ds(to, 16)]
            col = jnp.full((16,), h, jnp.int32)
            plsc.store_scatter(vbuf.at[b], [rows_t, col], exh, mask=mask_t)
            plsc.store_scatter(vbuf.at[b], [rows_t, col + 4], exh * a_t,
                               mask=mask_t)
            plsc.store_scatter(vbuf.at[b], [rows_t, col + 8], exh * b_t,
                               mask=mask_t)

        for j in range(8):
            sc_d[b].append(pltpu.async_copy(
                vbuf.at[b, pl.ds(j * IDXW, IDXW)],
                tsh.at[dbuf.at[ci * 8 + j]], s_sc[b], add=True))

    for b in range(2):
        for d in sc_d[b]:
            d.wait()

    plsc.subcore_barrier()
    pltpu.sync_copy(tsh.at[pl.ds(s * ZR, ZR)],
                    out_hbm.at[c, pl.ds(s * ZR, ZR)])


EB = 32000


def _tc_dense_body(co_ref, ea_ref, w5t_ref, hsa_ref, out_ref):
    coef5 = jnp.concatenate(
        [co_ref[...], ea_ref[...]], axis=0).astype(jnp.bfloat16)
    w5t = w5t_ref[...].astype(jnp.bfloat16)
    t0 = w5t[:, 0:1] * coef5[0:1, :] + w5t[:, 1:2] * coef5[1:2, :]
    t1 = w5t[:, 2:3] * coef5[2:3, :] + w5t[:, 3:4] * coef5[3:4, :]
    m = (t0 + t1) + w5t[:, 4:5] * coef5[4:5, :]
    g = jnp.maximum(m, jnp.bfloat16(0.2) * m)
    logits = lax.dot_general(hsa_ref[...].astype(jnp.bfloat16), g,
                             (((1,), (0,)), ((), ())),
                             preferred_element_type=jnp.float32)
    out_ref[...] = jnp.exp(logits)


_tc_dense = pl.pallas_call(
    _tc_dense_body,
    grid=(E // EB,),
    in_specs=[
        pl.BlockSpec((4, EB), lambda i: (0, i)),
        pl.BlockSpec((1, EB), lambda i: (0, i)),
        pl.BlockSpec((HC, 5), lambda i: (0, 0)),
        pl.BlockSpec((H, HC), lambda i: (0, 0)),
    ],
    out_specs=pl.BlockSpec((4, EB), lambda i: (0, i)),
    out_shape=jax.ShapeDtypeStruct((4, E), jnp.float32),
)


NB = 5000
NBLK = N // NB


def _tc_post_body(t_ref, wl_ref, gam_ref, bet_ref, y_ref, acc_ref, g_ref):
    i = pl.program_id(0)
    ts = t_ref[0] + t_ref[1]
    den = ts[:, 0:4] + 1e-16
    sa = ts[:, 4:8] / den
    sb = ts[:, 8:12] / den

    @pl.when(i < NBLK)
    def _phase0():
        m1 = jnp.sum(jnp.concatenate([sa, sb], axis=1), axis=0, keepdims=True)
        m2 = jnp.sum(jnp.concatenate([sa * sa, sb * sb], axis=1),
                     axis=0, keepdims=True)
        m3 = jnp.sum(jnp.concatenate([sa * sb, sa * sb], axis=1),
                     axis=0, keepdims=True)
        blk = jnp.concatenate([m1, m2, m3], axis=0)
        blk = jnp.concatenate([blk, jnp.zeros((3, 120), jnp.float32)], axis=1)
        blk = jnp.concatenate([blk, jnp.zeros((5, 128), jnp.float32)], axis=0)

        @pl.when(i == 0)
        def _():
            acc_ref[...] = jnp.zeros_like(acc_ref)

        acc_ref[...] += blk

        @pl.when(i == NBLK - 1)
        def _finalize():
            acc = acc_ref[...]
            inv_n = 1.0 / N
            m1f = acc[0:1, 0:8] * inv_n
            m2f = acc[1:2, 0:8] * inv_n
            m3f = acc[2:3, 0:8] * inv_n
            am, bm = m1f[:, 0:4], m1f[:, 4:8]
            var_a = m2f[:, 0:4] - am * am
            var_b = m2f[:, 4:8] - bm * bm
            cov = m3f[:, 0:4] - am * bm
            hs = (lax.broadcasted_iota(jnp.int32, (H, HC), 1) // C
                  == lax.broadcasted_iota(jnp.int32, (H, HC), 0)
                  ).astype(jnp.float32)

            def expand(z):
                return lax.dot_general(z, hs, (((1,), (0,)), ((), ())),
                                       preferred_element_type=jnp.float32,
                                       precision=lax.Precision.HIGHEST)

            u = wl_ref[0:1, :]
            v = wl_ref[1:2, :]
            var = (u * u * expand(var_a) + v * v * expand(var_b)
                   + 2.0 * u * v * expand(cov))
            sig = jnp.sqrt(var + 1e-5)
            p = gam_ref[...] * u / sig
            q = gam_ref[...] * v / sig
            const = bet_ref[...] - expand(am) * p - expand(bm) * q
            g_ref[...] = jnp.concatenate(
                [hs * p, hs * q, const, jnp.zeros((7, HC), jnp.float32)],
                axis=0)

    @pl.when(i >= NBLK)
    def _phase1():
        z = jnp.concatenate(
            [sa, sb, jnp.ones((NB, 1), jnp.float32),
             jnp.zeros((NB, 7), jnp.float32)], axis=1)
        y_ref[...] = lax.dot_general(z, g_ref[...], (((1,), (0,)), ((), ())),
                                     preferred_element_type=jnp.float32,
                                     precision=lax.Precision.HIGHEST)


def _t_map(i):
    return (0, jnp.where(i < NBLK, i, i - NBLK), 0)


_tc_post = pl.pallas_call(
    _tc_post_body,
    grid=(2 * NBLK,),
    in_specs=[
        pl.BlockSpec((NC, NB, 16), _t_map),
        pl.BlockSpec((2, HC), lambda i: (0, 0)),
        pl.BlockSpec((1, HC), lambda i: (0, 0)),
        pl.BlockSpec((1, HC), lambda i: (0, 0)),
    ],
    out_specs=pl.BlockSpec((NB, HC),
                           lambda i: (jnp.maximum(i - NBLK, 0), 0)),
    out_shape=jax.ShapeDtypeStruct((N, HC), jnp.float32),
    scratch_shapes=[pltpu.VMEM((8, 128), jnp.float32),
                    pltpu.VMEM((16, HC), jnp.float32)],
)


def kernel(x, edge_index, edge_attr, Wl, Wr, We, att, bias, gamma, beta):
    del bias
    src = edge_index[0]
    dst = edge_index[1]
    dst_r = dst.reshape(E // IDXW, IDXW)
    ea = edge_attr.reshape(1, E)
    w5t = jnp.stack([Wl[0], Wl[1], Wr[0], Wr[1], We[0]], axis=1)
    hmap = jnp.arange(HC, dtype=jnp.int32) // C
    hsa = jnp.where(hmap[None, :] == jnp.arange(H, dtype=jnp.int32)[:, None],
                    att.reshape(1, HC), 0.0)

    co = _sc_gather(x.reshape(-1), src, dst)
    ex = _tc_dense(co, ea, w5t, hsa)
    t = _sc_scatter(ex, co, dst_r)
    return _tc_post(t, Wl, gamma.reshape(1, HC), beta.reshape(1, HC))


# speedup vs baseline: 1.0642x; 1.0642x over previous
import functools

import jax
import jax.numpy as jnp
from jax import lax
from jax.experimental import pallas as pl
from jax.experimental.pallas import tpu as pltpu
from jax.experimental.pallas import tpu_sc as plsc

N = 50000
E = 800000
H = 4
C = 32
HC = 128

NC = 2
NS = 16
NW = NC * NS
W_PER = E // NW
CH = 1000
CHUNKS = W_PER // CH
FULLG = CH // 16
IDXW = 125
NP = 50048
ZR = NP // NS

_mesh = plsc.VectorSubcoreMesh(
    core_axis_name="c", subcore_axis_name="s", num_cores=NC, num_subcores=NS)
_sc_params = pltpu.CompilerParams(
    needs_layout_passes=False, use_tc_tiling_on_sc=False)


@functools.partial(
    pl.kernel,
    out_type=jax.ShapeDtypeStruct((4, E), jnp.float32),
    mesh=_mesh,
    scratch_types=[
        pltpu.VMEM((2 * N,), jnp.float32),
        pltpu.VMEM((2, CH), jnp.int32),
        pltpu.VMEM((2, CH), jnp.int32),
        pltpu.VMEM((2, 4, CH), jnp.float32),
        pltpu.SemaphoreType.DMA,
        pltpu.SemaphoreType.DMA,
        pltpu.SemaphoreType.DMA,
        pltpu.SemaphoreType.DMA,
        pltpu.SemaphoreType.DMA,
    ],
    compiler_params=_sc_params,
)
def _sc_gather(x_hbm, ei_hbm, out_hbm, xtab, sbuf, dbuf, cbuf,
               sx, si0, si1, so0, so1):
    c = lax.axis_index("c")
    s = lax.axis_index("s")
    wid = s * NC + c
    base = wid * W_PER
    s_in = (si0, si1)
    s_out = (so0, so1)

    xc = pltpu.async_copy(x_hbm, xtab, sx)

    def start_in(ci):
        b = ci & 1
        off = pl.multiple_of(base + ci * CH, CH)
        d1 = pltpu.async_copy(ei_hbm.at[0, pl.ds(off, CH)], sbuf.at[b],
                              s_in[b])
        d2 = pltpu.async_copy(ei_hbm.at[1, pl.ds(off, CH)], dbuf.at[b],
                              s_in[b])
        return (d1, d2)

    in_d = [None, None]
    in_d[0] = start_in(0)
    xc.wait()
    out_d = [None, None]

    for ci in range(CHUNKS):
        b = ci & 1
        if ci + 1 < CHUNKS:
            in_d[1 - b] = start_in(ci + 1)
        in_d[b][0].wait()
        in_d[b][1].wait()
        if out_d[b] is not None:
            for d in out_d[b]:
                d.wait()

        @pl.loop(0, FULLG)
        def _grp(g, b=b):
            si = sbuf[b, pl.ds(g * 16, 16)] * 2
            di = dbuf[b, pl.ds(g * 16, 16)] * 2
            cbuf[b, 0, pl.ds(g * 16, 16)] = plsc.load_gather(xtab, [si])
            cbuf[b, 1, pl.ds(g * 16, 16)] = plsc.load_gather(xtab, [si + 1])
            cbuf[b, 2, pl.ds(g * 16, 16)] = plsc.load_gather(xtab, [di])
            cbuf[b, 3, pl.ds(g * 16, 16)] = plsc.load_gather(xtab, [di + 1])

        to = CH - 16
        sit = sbuf[b, pl.ds(to, 16)] * 2
        dit = dbuf[b, pl.ds(to, 16)] * 2
        cbuf[b, 0, pl.ds(to, 16)] = plsc.load_gather(xtab, [sit])
        cbuf[b, 1, pl.ds(to, 16)] = plsc.load_gather(xtab, [sit + 1])
        cbuf[b, 2, pl.ds(to, 16)] = plsc.load_gather(xtab, [dit])
        cbuf[b, 3, pl.ds(to, 16)] = plsc.load_gather(xtab, [dit + 1])

        off = pl.multiple_of(base + ci * CH, CH)
        out_d[b] = [
            pltpu.async_copy(cbuf.at[b, r], out_hbm.at[r, pl.ds(off, CH)],
                             s_out[b])
            for r in range(4)]

    for bb in range(2):
        if out_d[bb] is not None:
            for d in out_d[bb]:
                d.wait()


@functools.partial(
    pl.kernel,
    out_type=jax.ShapeDtypeStruct((NC, NP, 16), jnp.float32),
    mesh=_mesh,
    scratch_types=[
        pltpu.VMEM_SHARED((NP, 16), jnp.float32),
        pltpu.VMEM((CHUNKS * 8, IDXW), jnp.int32),
        pltpu.VMEM((2, 4, CH), jnp.float32),
        pltpu.VMEM((2, 2, CH), jnp.float32),
        pltpu.VMEM((2, CH, 16), jnp.float32),
        pltpu.SemaphoreType.DMA,
        pltpu.SemaphoreType.DMA,
        pltpu.SemaphoreType.DMA,
        pltpu.SemaphoreType.DMA,
    ],
    compiler_params=_sc_params,
)
def _sc_scatter(ex_hbm, co_hbm, dst_hbm, out_hbm,
                tsh, dbuf, exbuf, abbuf, vbuf, si0, si1, sc0, sc1):
    c = lax.axis_index("c")
    s = lax.axis_index("s")
    wid = s * NC + c
    base = wid * W_PER
    s_in = (si0, si1)
    s_sc = (sc0, sc1)

    dst_pf = pltpu.async_copy(
        dst_hbm.at[pl.ds(pl.multiple_of(base // IDXW, 8), CHUNKS * 8)],
        dbuf, si0)

    @pl.loop(0, CH)
    def _z(i):
        vbuf[0, i, :] = jnp.zeros((16,), jnp.float32)
        vbuf[1, i, :] = jnp.zeros((16,), jnp.float32)

    row0 = pl.multiple_of(s * ZR, 8)
    for r in range(3):
        pltpu.sync_copy(vbuf.at[0],
                        tsh.at[pl.ds(row0 + r * CH, CH)])
    pltpu.sync_copy(vbuf.at[0, pl.ds(0, ZR - 3 * CH)],
                    tsh.at[pl.ds(row0 + 3 * CH, ZR - 3 * CH)])
    plsc.subcore_barrier()

    def start_in(ci):
        b = ci & 1
        off = pl.multiple_of(base + ci * CH, CH)
        ds = [pltpu.async_copy(ex_hbm.at[h, pl.ds(off, CH)],
                               exbuf.at[b, h], s_in[b]) for h in range(4)]
        ds += [pltpu.async_copy(co_hbm.at[r, pl.ds(off, CH)],
                                abbuf.at[b, r], s_in[b]) for r in range(2)]
        return ds

    in_d = [None, None]
    in_d[0] = start_in(0)
    dst_pf.wait()
    sc_d = [[], []]

    for ci in range(CHUNKS):
        b = ci & 1
        if ci + 1 < CHUNKS:
            in_d[1 - b] = start_in(ci + 1)
        for d in in_d[b]:
            d.wait()
        for d in sc_d[b]:
            d.wait()
        sc_d[b] = []

        @pl.loop(0, FULLG)
        def _grp(g, b=b):
            rows = g * 16 + lax.iota(jnp.int32, 16)
            a = abbuf[b, 0, pl.ds(g * 16, 16)]
            bb = abbuf[b, 1, pl.ds(g * 16, 16)]
            for h in range(H):
                exh = exbuf[b, h, pl.ds(g * 16, 16)]
                col = jnp.full((16,), h, jnp.int32)
                plsc.store_scatter(vbuf.at[b], [rows, col], exh)
                plsc.store_scatter(vbuf.at[b], [rows, col + 4], exh * a)
                plsc.store_scatter(vbuf.at[b], [rows, col + 8], exh * bb)

        to = CH - 16
        rows_t = to + lax.iota(jnp.int32, 16)
        mask_t = lax.iota(jnp.int32, 16) >= 8
        a_t = abbuf[b, 0, pl.ds(to, 16)]
        b_t = abbuf[b, 1, pl.ds(to, 16)]
        for h in range(H):
            exh = exbuf[b, h, pl.ds(to, 16)]
            col = jnp.full((16,), h, jnp.int32)
            plsc.store_scatter(vbuf.at[b], [rows_t, col], exh, mask=mask_t)
            plsc.store_scatter(vbuf.at[b], [rows_t, col + 4], exh * a_t,
                               mask=mask_t)
            plsc.store_scatter(vbuf.at[b], [rows_t, col + 8], exh * b_t,
                               mask=mask_t)

        for j in range(8):
            sc_d[b].append(pltpu.async_copy(
                vbuf.at[b, pl.ds(j * IDXW, IDXW)],
                tsh.at[dbuf.at[ci * 8 + j]], s_sc[b], add=True))

    for b in range(2):
        for d in sc_d[b]:
            d.wait()

    plsc.subcore_barrier()
    pltpu.sync_copy(tsh.at[pl.ds(s * ZR, ZR)],
                    out_hbm.at[c, pl.ds(s * ZR, ZR)])


EB = 16000


def _tc_dense_body(co_ref, ea_ref, w5t_ref, hsa_ref, out_ref):
    coef5 = jnp.concatenate(
        [co_ref[...], ea_ref[...]], axis=0).astype(jnp.bfloat16)
    w5t = w5t_ref[...].astype(jnp.bfloat16)
    t0 = w5t[:, 0:1] * coef5[0:1, :] + w5t[:, 1:2] * coef5[1:2, :]
    t1 = w5t[:, 2:3] * coef5[2:3, :] + w5t[:, 3:4] * coef5[3:4, :]
    m = (t0 + t1) + w5t[:, 4:5] * coef5[4:5, :]
    g = jnp.maximum(m, jnp.bfloat16(0.2) * m)
    logits = lax.dot_general(hsa_ref[...].astype(jnp.bfloat16), g,
                             (((1,), (0,)), ((), ())),
                             preferred_element_type=jnp.float32)
    out_ref[...] = jnp.exp(logits)


_tc_dense = pl.pallas_call(
    _tc_dense_body,
    grid=(E // EB,),
    in_specs=[
        pl.BlockSpec((4, EB), lambda i: (0, i)),
        pl.BlockSpec((1, EB), lambda i: (0, i)),
        pl.BlockSpec((HC, 5), lambda i: (0, 0)),
        pl.BlockSpec((H, HC), lambda i: (0, 0)),
    ],
    out_specs=pl.BlockSpec((4, EB), lambda i: (0, i)),
    out_shape=jax.ShapeDtypeStruct((4, E), jnp.float32),
)


NB = 2000
NBLK = N // NB


def _tc_post_body(t_ref, wl_ref, gam_ref, bet_ref, y_ref, acc_ref, g_ref):
    i = pl.program_id(0)
    ts = t_ref[0] + t_ref[1]
    den = ts[:, 0:4] + 1e-16
    sa = ts[:, 4:8] / den
    sb = ts[:, 8:12] / den

    @pl.when(i < NBLK)
    def _phase0():
        m1 = jnp.sum(jnp.concatenate([sa, sb], axis=1), axis=0, keepdims=True)
        m2 = jnp.sum(jnp.concatenate([sa * sa, sb * sb], axis=1),
                     axis=0, keepdims=True)
        m3 = jnp.sum(jnp.concatenate([sa * sb, sa * sb], axis=1),
                     axis=0, keepdims=True)
        blk = jnp.concatenate([m1, m2, m3], axis=0)
        blk = jnp.concatenate([blk, jnp.zeros((3, 120), jnp.float32)], axis=1)
        blk = jnp.concatenate([blk, jnp.zeros((5, 128), jnp.float32)], axis=0)

        @pl.when(i == 0)
        def _():
            acc_ref[...] = jnp.zeros_like(acc_ref)

        acc_ref[...] += blk

        @pl.when(i == NBLK - 1)
        def _finalize():
            acc = acc_ref[...]
            inv_n = 1.0 / N
            m1f = acc[0:1, 0:8] * inv_n
            m2f = acc[1:2, 0:8] * inv_n
            m3f = acc[2:3, 0:8] * inv_n
            am, bm = m1f[:, 0:4], m1f[:, 4:8]
            var_a = m2f[:, 0:4] - am * am
            var_b = m2f[:, 4:8] - bm * bm
            cov = m3f[:, 0:4] - am * bm
            hs = (lax.broadcasted_iota(jnp.int32, (H, HC), 1) // C
                  == lax.broadcasted_iota(jnp.int32, (H, HC), 0)
                  ).astype(jnp.float32)

            def expand(z):
                return lax.dot_general(z, hs, (((1,), (0,)), ((), ())),
                                       preferred_element_type=jnp.float32,
                                       precision=lax.Precision.HIGHEST)

            u = wl_ref[0:1, :]
            v = wl_ref[1:2, :]
            var = (u * u * expand(var_a) + v * v * expand(var_b)
                   + 2.0 * u * v * expand(cov))
            sig = jnp.sqrt(var + 1e-5)
            p = gam_ref[...] * u / sig
            q = gam_ref[...] * v / sig
            const = bet_ref[...] - expand(am) * p - expand(bm) * q
            g_ref[...] = jnp.concatenate(
                [hs * p, hs * q, const, jnp.zeros((7, HC), jnp.float32)],
                axis=0)

    @pl.when(i >= NBLK)
    def _phase1():
        z = jnp.concatenate(
            [sa, sb, jnp.ones((NB, 1), jnp.float32),
             jnp.zeros((NB, 7), jnp.float32)], axis=1)
        y_ref[...] = lax.dot_general(z, g_ref[...], (((1,), (0,)), ((), ())),
                                     preferred_element_type=jnp.float32,
                                     precision=lax.Precision.HIGHEST)


def _t_map(i):
    return (0, jnp.where(i < NBLK, i, i - NBLK), 0)


_tc_post = pl.pallas_call(
    _tc_post_body,
    grid=(2 * NBLK,),
    in_specs=[
        pl.BlockSpec((NC, NB, 16), _t_map),
        pl.BlockSpec((2, HC), lambda i: (0, 0)),
        pl.BlockSpec((1, HC), lambda i: (0, 0)),
        pl.BlockSpec((1, HC), lambda i: (0, 0)),
    ],
    out_specs=pl.BlockSpec((NB, HC),
                           lambda i: (jnp.maximum(i - NBLK, 0), 0)),
    out_shape=jax.ShapeDtypeStruct((N, HC), jnp.float32),
    scratch_shapes=[pltpu.VMEM((8, 128), jnp.float32),
                    pltpu.VMEM((16, HC), jnp.float32)],
)


def kernel(x, edge_index, edge_attr, Wl, Wr, We, att, bias, gamma, beta):
    del bias
    dst_r = edge_index[1].reshape(E // IDXW, IDXW)
    ea = edge_attr.reshape(1, E)
    w5t = jnp.stack([Wl[0], Wl[1], Wr[0], Wr[1], We[0]], axis=1)
    hmap = jnp.arange(HC, dtype=jnp.int32) // C
    hsa = jnp.where(hmap[None, :] == jnp.arange(H, dtype=jnp.int32)[:, None],
                    att.reshape(1, HC), 0.0)

    co = _sc_gather(x.reshape(-1), edge_index)
    ex = _tc_dense(co, ea, w5t, hsa)
    t = _sc_scatter(ex, co, dst_r)
    return _tc_post(t, Wl, gamma.reshape(1, HC), beta.reshape(1, HC))
